# Initial kernel scaffold; baseline (speedup 1.0000x reference)
import jax
import jax.numpy as jnp
from jax.experimental import pallas as pl


def kernel(slates_item_categorical, slates_item_indexes, responses, emb_tables):
    raise NotImplementedError("write your pallas kernel here")


# speedup vs baseline: 5.6487x; 5.6487x over previous
import functools

import jax
import jax.numpy as jnp
from jax.experimental import pallas as pl


def _cumsum(x, axis):
    n = x.shape[axis]
    k = 1
    while k < n:
        pad = jnp.zeros_like(jax.lax.slice_in_dim(x, 0, k, axis=axis))
        shifted = jnp.concatenate(
            [pad, jax.lax.slice_in_dim(x, 0, n - k, axis=axis)], axis=axis
        )
        x = x + shifted
        k *= 2
    return x


def _body(cat_ref, resp_ref, tab_ref, oe_ref, oc_ref, *, BB, S, L, F, VP, D):
    N = BB * S * L
    cat = cat_ref[...]
    col = jax.lax.broadcasted_iota(jnp.int32, (N, F * VP), 1)
    oh = (col == cat[:, 0][:, None]).astype(jnp.float32)
    for i in range(1, F):
        oh = oh + (col == (cat[:, i][:, None] + i * VP)).astype(jnp.float32)
    emb = jnp.dot(oh, tab_ref[...], preferred_element_type=jnp.float32)
    oe_ref[...] = emb

    w = resp_ref[...]
    e4 = emb.reshape(BB, S, L, D)
    ws = jnp.sum(e4 * w[..., None], axis=2)
    wsum = jnp.sum(w, axis=2)
    cons = _cumsum(ws, axis=1) - ws
    num = _cumsum(wsum, axis=1) - wsum
    denom = jnp.maximum(num, 1.0)[..., None]
    cons = jnp.where((num > 0)[..., None], cons / denom, cons)
    oc_ref[...] = cons.reshape(BB * S, D)


def kernel(slates_item_categorical, slates_item_indexes, responses, emb_tables):
    del slates_item_indexes
    B, S, L, F = slates_item_categorical.shape
    _, V, DC = emb_tables.shape
    D = F * DC
    VP = 128

    tab = jnp.zeros((F * VP, D), jnp.float32)
    for i in range(F):
        tab = tab.at[i * VP : i * VP + V, i * DC : (i + 1) * DC].set(emb_tables[i])

    BB = 16
    grid = B // BB
    N = BB * S * L

    cat2 = slates_item_categorical.reshape(B * S * L, F)
    respf = responses.astype(jnp.float32)

    oe, oc = pl.pallas_call(
        functools.partial(_body, BB=BB, S=S, L=L, F=F, VP=VP, D=D),
        grid=(grid,),
        in_specs=[
            pl.BlockSpec((N, F), lambda i: (i, 0)),
            pl.BlockSpec((BB, S, L), lambda i: (i, 0, 0)),
            pl.BlockSpec((F * VP, D), lambda i: (0, 0)),
        ],
        out_specs=[
            pl.BlockSpec((N, D), lambda i: (i, 0)),
            pl.BlockSpec((BB * S, D), lambda i: (i, 0)),
        ],
        out_shape=[
            jax.ShapeDtypeStruct((B * S * L, D), jnp.float32),
            jax.ShapeDtypeStruct((B * S, D), jnp.float32),
        ],
    )(cat2, respf, tab)

    return oe.reshape(B, S, L, D), oc.reshape(B, S, D)


# speedup vs baseline: 5.6561x; 1.0013x over previous
import functools

import jax
import jax.numpy as jnp
from jax.experimental import pallas as pl


def _cumsum(x, axis):
    n = x.shape[axis]
    k = 1
    while k < n:
        pad = jnp.zeros_like(jax.lax.slice_in_dim(x, 0, k, axis=axis))
        shifted = jnp.concatenate(
            [pad, jax.lax.slice_in_dim(x, 0, n - k, axis=axis)], axis=axis
        )
        x = x + shifted
        k *= 2
    return x


def _body(cat_ref, resp_ref, tab_ref, oe_ref, oc_ref, *, BB, S, L, F, VP, D):
    N = BB * S * L
    cat = cat_ref[...]
    col = jax.lax.broadcasted_iota(jnp.int32, (N, F * VP), 1)
    oh = (col == cat[:, 0][:, None]).astype(jnp.bfloat16)
    for i in range(1, F):
        oh = oh + (col == (cat[:, i][:, None] + i * VP)).astype(jnp.bfloat16)
    emb = jnp.dot(oh, tab_ref[...], preferred_element_type=jnp.float32)
    oe_ref[...] = emb

    w = resp_ref[...]
    e4 = emb.reshape(BB, S, L, D)
    ws = jnp.sum(e4 * w[..., None], axis=2)
    wsum = jnp.sum(w, axis=2)
    cons = _cumsum(ws, axis=1) - ws
    num = _cumsum(wsum, axis=1) - wsum
    denom = jnp.maximum(num, 1.0)[..., None]
    cons = jnp.where((num > 0)[..., None], cons / denom, cons)
    oc_ref[...] = cons.reshape(BB * S, D)


def kernel(slates_item_categorical, slates_item_indexes, responses, emb_tables):
    del slates_item_indexes
    B, S, L, F = slates_item_categorical.shape
    _, V, DC = emb_tables.shape
    D = F * DC
    VP = 128

    tab = jnp.zeros((F * VP, D), jnp.float32)
    for i in range(F):
        tab = tab.at[i * VP : i * VP + V, i * DC : (i + 1) * DC].set(emb_tables[i])
    tab = tab.astype(jnp.bfloat16)

    BB = 16
    grid = B // BB
    N = BB * S * L

    cat2 = slates_item_categorical.reshape(B * S * L, F)
    respf = responses.astype(jnp.float32)

    oe, oc = pl.pallas_call(
        functools.partial(_body, BB=BB, S=S, L=L, F=F, VP=VP, D=D),
        grid=(grid,),
        in_specs=[
            pl.BlockSpec((N, F), lambda i: (i, 0)),
            pl.BlockSpec((BB, S, L), lambda i: (i, 0, 0)),
            pl.BlockSpec((F * VP, D), lambda i: (0, 0)),
        ],
        out_specs=[
            pl.BlockSpec((N, D), lambda i: (i, 0)),
            pl.BlockSpec((BB * S, D), lambda i: (i, 0)),
        ],
        out_shape=[
            jax.ShapeDtypeStruct((B * S * L, D), jnp.float32),
            jax.ShapeDtypeStruct((B * S, D), jnp.float32),
        ],
    )(cat2, respf, tab)

    return oe.reshape(B, S, L, D), oc.reshape(B, S, D)


# speedup vs baseline: 6.6901x; 1.1828x over previous
import functools

import jax
import jax.numpy as jnp
from jax.experimental import pallas as pl


def _cumsum(x, axis):
    n = x.shape[axis]
    k = 1
    while k < n:
        pad = jnp.zeros_like(jax.lax.slice_in_dim(x, 0, k, axis=axis))
        shifted = jnp.concatenate(
            [pad, jax.lax.slice_in_dim(x, 0, n - k, axis=axis)], axis=axis
        )
        x = x + shifted
        k *= 2
    return x


def _body(cat_ref, resp_ref, wf_ref, q_ref, tab_ref, oe_ref, oc_ref, *, BB, S, L, F, VP, D):
    N = BB * S * L
    cat = cat_ref[...]
    col = jax.lax.broadcasted_iota(jnp.int32, (N, VP), 1)
    emb = jnp.zeros((N, D), jnp.float32)
    for i in range(F):
        ohi = (col == cat[:, i][:, None]).astype(jnp.bfloat16)
        emb = emb + jnp.dot(ohi, tab_ref[i], preferred_element_type=jnp.float32)
    oe_ref[...] = emb

    wemb = (emb * wf_ref[...]).astype(jnp.bfloat16)
    cons = jnp.dot(q_ref[...], wemb, preferred_element_type=jnp.float32)

    w = resp_ref[...]
    num = _cumsum(jnp.sum(w, axis=2), axis=1)
    num = num - jnp.sum(w, axis=2)
    denom = jnp.maximum(num, 1.0)[..., None]
    c3 = cons.reshape(BB, S, D)
    c3 = jnp.where((num > 0)[..., None], c3 / denom, c3)
    oc_ref[...] = c3.reshape(BB * S, D)


def kernel(slates_item_categorical, slates_item_indexes, responses, emb_tables):
    del slates_item_indexes
    B, S, L, F = slates_item_categorical.shape
    _, V, DC = emb_tables.shape
    D = F * DC
    VP = 128

    tab = jnp.zeros((F, VP, D), jnp.float32)
    for i in range(F):
        tab = tab.at[i, :V, i * DC : (i + 1) * DC].set(emb_tables[i])
    tab = tab.astype(jnp.bfloat16)

    BB = 16
    grid = B // BB
    N = BB * S * L

    r = jnp.arange(BB * S, dtype=jnp.int32)
    n = jnp.arange(N, dtype=jnp.int32)
    q = ((n[None, :] // (S * L)) == (r[:, None] // S)) & (
        (n[None, :] % (S * L)) < (r[:, None] % S) * L
    )
    q = q.astype(jnp.bfloat16)

    cat2 = slates_item_categorical.reshape(B * S * L, F)
    respf = responses.astype(jnp.float32)
    wflat = respf.reshape(B * S * L, 1)

    oe, oc = pl.pallas_call(
        functools.partial(_body, BB=BB, S=S, L=L, F=F, VP=VP, D=D),
        grid=(grid,),
        in_specs=[
            pl.BlockSpec((N, F), lambda i: (i, 0)),
            pl.BlockSpec((BB, S, L), lambda i: (i, 0, 0)),
            pl.BlockSpec((N, 1), lambda i: (i, 0)),
            pl.BlockSpec((BB * S, N), lambda i: (0, 0)),
            pl.BlockSpec((F, VP, D), lambda i: (0, 0, 0)),
        ],
        out_specs=[
            pl.BlockSpec((N, D), lambda i: (i, 0)),
            pl.BlockSpec((BB * S, D), lambda i: (i, 0)),
        ],
        out_shape=[
            jax.ShapeDtypeStruct((B * S * L, D), jnp.float32),
            jax.ShapeDtypeStruct((B * S, D), jnp.float32),
        ],
    )(cat2, respf, wflat, q, tab)

    return oe.reshape(B, S, L, D), oc.reshape(B, S, D)


# speedup vs baseline: 8.5252x; 1.2743x over previous
import functools

import jax
import jax.numpy as jnp
from jax.experimental import pallas as pl


def _cumsum(x, axis):
    n = x.shape[axis]
    k = 1
    while k < n:
        pad = jnp.zeros_like(jax.lax.slice_in_dim(x, 0, k, axis=axis))
        shifted = jnp.concatenate(
            [pad, jax.lax.slice_in_dim(x, 0, n - k, axis=axis)], axis=axis
        )
        x = x + shifted
        k *= 2
    return x


def _body(cat_ref, resp_ref, wf_ref, q_ref, tab_ref, oe_ref, oc_ref, *, BB, S, L, F, VP, D):
    N = BB * S * L
    cat = cat_ref[...]
    col = jax.lax.broadcasted_iota(jnp.int32, (N, VP), 1)
    emb = jnp.zeros((N, D), jnp.float32)
    for i in range(F):
        ohi = (col == cat[:, i][:, None]).astype(jnp.bfloat16)
        emb = emb + jnp.dot(ohi, tab_ref[i], preferred_element_type=jnp.float32)
    oe_ref[...] = emb.reshape(BB, S, L, D)

    wemb = (emb * wf_ref[...]).astype(jnp.bfloat16)
    cons = jnp.dot(q_ref[...], wemb, preferred_element_type=jnp.float32)

    w = resp_ref[...]
    num = _cumsum(jnp.sum(w, axis=2), axis=1)
    num = num - jnp.sum(w, axis=2)
    denom = jnp.maximum(num, 1.0)[..., None]
    c3 = cons.reshape(BB, S, D)
    oc_ref[...] = jnp.where((num > 0)[..., None], c3 / denom, c3)


def kernel(slates_item_categorical, slates_item_indexes, responses, emb_tables):
    del slates_item_indexes
    B, S, L, F = slates_item_categorical.shape
    _, V, DC = emb_tables.shape
    D = F * DC
    VP = 128

    tab = jnp.zeros((F, VP, D), jnp.float32)
    for i in range(F):
        tab = tab.at[i, :V, i * DC : (i + 1) * DC].set(emb_tables[i])
    tab = tab.astype(jnp.bfloat16)

    BB = 16
    grid = B // BB
    N = BB * S * L

    r = jnp.arange(BB * S, dtype=jnp.int32)
    n = jnp.arange(N, dtype=jnp.int32)
    q = ((n[None, :] // (S * L)) == (r[:, None] // S)) & (
        (n[None, :] % (S * L)) < (r[:, None] % S) * L
    )
    q = q.astype(jnp.bfloat16)

    cat2 = slates_item_categorical.reshape(B * S * L, F)
    respf = responses.astype(jnp.float32)
    wflat = respf.reshape(B * S * L, 1)

    oe, oc = pl.pallas_call(
        functools.partial(_body, BB=BB, S=S, L=L, F=F, VP=VP, D=D),
        grid=(grid,),
        in_specs=[
            pl.BlockSpec((N, F), lambda i: (i, 0)),
            pl.BlockSpec((BB, S, L), lambda i: (i, 0, 0)),
            pl.BlockSpec((N, 1), lambda i: (i, 0)),
            pl.BlockSpec((BB * S, N), lambda i: (0, 0)),
            pl.BlockSpec((F, VP, D), lambda i: (0, 0, 0)),
        ],
        out_specs=[
            pl.BlockSpec((BB, S, L, D), lambda i: (i, 0, 0, 0)),
            pl.BlockSpec((BB, S, D), lambda i: (i, 0, 0)),
        ],
        out_shape=[
            jax.ShapeDtypeStruct((B, S, L, D), jnp.float32),
            jax.ShapeDtypeStruct((B, S, D), jnp.float32),
        ],
    )(cat2, respf, wflat, q, tab)

    return oe, oc
